# Initial kernel scaffold; baseline (speedup 1.0000x reference)
import jax
import jax.numpy as jnp
from jax.experimental import pallas as pl


def kernel(inp):
    raise NotImplementedError("write your pallas kernel here")


# speedup vs baseline: 5.6973x; 5.6973x over previous
import jax
import jax.numpy as jnp
from jax.experimental import pallas as pl

_N_FEATURES = 8
_FREQUENCY = 0.01
_GAIN = 1.0


def _season_params(num_cols):
    key = jax.random.key(42)
    key, kf = jax.random.split(key)
    features = jax.random.randint(kf, (_N_FEATURES,), 0, num_cols)
    freqs, gains = [], []
    for _ in range(_N_FEATURES):
        key, k1, k2 = jax.random.split(key, 3)
        freqs.append(jax.random.uniform(k1, (), dtype=jnp.float32) * _FREQUENCY)
        gains.append(jax.random.uniform(k2, (), dtype=jnp.float32) * _GAIN)
    return features, jnp.stack(freqs), jnp.stack(gains)


def _body(feat_ref, freq_ref, gain_ref, x_ref, o_ref, *, rows_total):
    i = pl.program_id(0)
    blk_rows, num_cols = o_ref.shape
    row = (
        jax.lax.broadcasted_iota(jnp.int32, (blk_rows, 1), 0) + i * blk_rows
    ).astype(jnp.float32)
    t = row * (1.0 / rows_total)
    x = t * 2.0 * 3.14 * freq_ref[0, :]
    season = gain_ref[0, :] * jnp.sin(x)
    col = jax.lax.broadcasted_iota(
        jnp.int32, (_N_FEATURES, num_cols), 1
    ).astype(jnp.float32)
    onehot = (col == feat_ref[0, :].reshape(_N_FEATURES, 1)).astype(jnp.float32)
    o_ref[...] = x_ref[...] + jax.lax.dot(
        season, onehot, preferred_element_type=jnp.float32
    )


def kernel(inp):
    rows, cols = inp.shape
    features, freqs, gains = _season_params(cols)
    featf = features.astype(jnp.float32).reshape(1, _N_FEATURES)
    freqs = freqs.reshape(1, _N_FEATURES)
    gains = gains.reshape(1, _N_FEATURES)

    blk = 512
    grid = rows // blk
    import functools

    return pl.pallas_call(
        functools.partial(_body, rows_total=rows),
        grid=(grid,),
        in_specs=[
            pl.BlockSpec((1, _N_FEATURES), lambda i: (0, 0)),
            pl.BlockSpec((1, _N_FEATURES), lambda i: (0, 0)),
            pl.BlockSpec((1, _N_FEATURES), lambda i: (0, 0)),
            pl.BlockSpec((blk, cols), lambda i: (i, 0)),
        ],
        out_specs=pl.BlockSpec((blk, cols), lambda i: (i, 0)),
        out_shape=jax.ShapeDtypeStruct((rows, cols), jnp.float32),
    )(featf, freqs, gains, inp)


# speedup vs baseline: 6.6870x; 1.1737x over previous
import jax
import jax.numpy as jnp
from jax.experimental import pallas as pl

_N_FEATURES = 8
_FREQUENCY = 0.01
_GAIN = 1.0


def _season_params(num_cols):
    key = jax.random.key(42)
    key, kf = jax.random.split(key)
    features = jax.random.randint(kf, (_N_FEATURES,), 0, num_cols)
    freqs, gains = [], []
    for _ in range(_N_FEATURES):
        key, k1, k2 = jax.random.split(key, 3)
        freqs.append(jax.random.uniform(k1, (), dtype=jnp.float32) * _FREQUENCY)
        gains.append(jax.random.uniform(k2, (), dtype=jnp.float32) * _GAIN)
    return features, jnp.stack(freqs), jnp.stack(gains)


def _body(feat_ref, freq_ref, gain_ref, x_ref, o_ref, *, rows_total):
    i = pl.program_id(0)
    blk_rows, num_cols = o_ref.shape
    row = (
        jax.lax.broadcasted_iota(jnp.int32, (blk_rows, 1), 0) + i * blk_rows
    ).astype(jnp.float32)
    t = row * (1.0 / rows_total)
    x = t * 2.0 * 3.14 * freq_ref[0, :]
    season = gain_ref[0, :] * jnp.sin(x)
    col = jax.lax.broadcasted_iota(
        jnp.int32, (_N_FEATURES, num_cols), 1
    ).astype(jnp.float32)
    onehot = (col == feat_ref[0, :].reshape(_N_FEATURES, 1)).astype(jnp.float32)
    o_ref[...] = x_ref[...] + jax.lax.dot(
        season, onehot, preferred_element_type=jnp.float32
    )


def kernel(inp):
    rows, cols = inp.shape
    features, freqs, gains = _season_params(cols)
    featf = features.astype(jnp.float32).reshape(1, _N_FEATURES)
    freqs = freqs.reshape(1, _N_FEATURES)
    gains = gains.reshape(1, _N_FEATURES)

    blk = 2048
    grid = rows // blk
    import functools

    return pl.pallas_call(
        functools.partial(_body, rows_total=rows),
        grid=(grid,),
        in_specs=[
            pl.BlockSpec((1, _N_FEATURES), lambda i: (0, 0)),
            pl.BlockSpec((1, _N_FEATURES), lambda i: (0, 0)),
            pl.BlockSpec((1, _N_FEATURES), lambda i: (0, 0)),
            pl.BlockSpec((blk, cols), lambda i: (i, 0)),
        ],
        out_specs=pl.BlockSpec((blk, cols), lambda i: (i, 0)),
        out_shape=jax.ShapeDtypeStruct((rows, cols), jnp.float32),
    )(featf, freqs, gains, inp)


# speedup vs baseline: 6.6991x; 1.0018x over previous
import jax
import jax.numpy as jnp
from jax.experimental import pallas as pl

_N_FEATURES = 8
_FREQUENCY = 0.01
_GAIN = 1.0


def _season_params(num_cols):
    key = jax.random.key(42)
    key, kf = jax.random.split(key)
    features = jax.random.randint(kf, (_N_FEATURES,), 0, num_cols)
    freqs, gains = [], []
    for _ in range(_N_FEATURES):
        key, k1, k2 = jax.random.split(key, 3)
        freqs.append(jax.random.uniform(k1, (), dtype=jnp.float32) * _FREQUENCY)
        gains.append(jax.random.uniform(k2, (), dtype=jnp.float32) * _GAIN)
    return features, jnp.stack(freqs), jnp.stack(gains)


def _body(feat_ref, freq_ref, gain_ref, x_ref, o_ref, *, rows_total):
    i = pl.program_id(0)
    blk_rows, num_cols = o_ref.shape
    row = (
        jax.lax.broadcasted_iota(jnp.int32, (blk_rows, 1), 0) + i * blk_rows
    ).astype(jnp.float32)
    t = row * (1.0 / rows_total)
    x = t * 2.0 * 3.14 * freq_ref[0, :]
    season = gain_ref[0, :] * jnp.sin(x)
    col = jax.lax.broadcasted_iota(
        jnp.int32, (_N_FEATURES, num_cols), 1
    ).astype(jnp.float32)
    onehot = (col == feat_ref[0, :].reshape(_N_FEATURES, 1)).astype(jnp.float32)
    o_ref[...] = x_ref[...] + jax.lax.dot(
        season, onehot, preferred_element_type=jnp.float32
    )


def kernel(inp):
    rows, cols = inp.shape
    features, freqs, gains = _season_params(cols)
    featf = features.astype(jnp.float32).reshape(1, _N_FEATURES)
    freqs = freqs.reshape(1, _N_FEATURES)
    gains = gains.reshape(1, _N_FEATURES)

    blk = 4096
    grid = rows // blk
    import functools

    return pl.pallas_call(
        functools.partial(_body, rows_total=rows),
        grid=(grid,),
        in_specs=[
            pl.BlockSpec((1, _N_FEATURES), lambda i: (0, 0)),
            pl.BlockSpec((1, _N_FEATURES), lambda i: (0, 0)),
            pl.BlockSpec((1, _N_FEATURES), lambda i: (0, 0)),
            pl.BlockSpec((blk, cols), lambda i: (i, 0)),
        ],
        out_specs=pl.BlockSpec((blk, cols), lambda i: (i, 0)),
        out_shape=jax.ShapeDtypeStruct((rows, cols), jnp.float32),
    )(featf, freqs, gains, inp)


# speedup vs baseline: 7.9073x; 1.1803x over previous
import jax
import jax.numpy as jnp
from jax.experimental import pallas as pl

_N_FEATURES = 8
_FREQUENCY = 0.01
_GAIN = 1.0


def _season_params(num_cols):
    key = jax.random.key(42)
    key, kf = jax.random.split(key)
    features = jax.random.randint(kf, (_N_FEATURES,), 0, num_cols)
    freqs, gains = [], []
    for _ in range(_N_FEATURES):
        key, k1, k2 = jax.random.split(key, 3)
        freqs.append(jax.random.uniform(k1, (), dtype=jnp.float32) * _FREQUENCY)
        gains.append(jax.random.uniform(k2, (), dtype=jnp.float32) * _GAIN)
    return features, jnp.stack(freqs), jnp.stack(gains)


def _body(feat_ref, freq_ref, gain_ref, x_ref, o_ref, *, rows_total):
    i = pl.program_id(0)
    blk_rows, num_cols = o_ref.shape
    row = (
        jax.lax.broadcasted_iota(jnp.int32, (_N_FEATURES, blk_rows), 1)
        + i * blk_rows
    ).astype(jnp.float32)
    t = row * (1.0 / rows_total)
    x = t * 2.0 * 3.14 * freq_ref[...]
    season = gain_ref[...] * jnp.sin(x)
    col = jax.lax.broadcasted_iota(
        jnp.int32, (_N_FEATURES, num_cols), 1
    ).astype(jnp.float32)
    onehot = (col == feat_ref[...]).astype(jnp.float32)
    o_ref[...] = x_ref[...] + jax.lax.dot_general(
        season,
        onehot,
        (((0,), (0,)), ((), ())),
        preferred_element_type=jnp.float32,
    )


def kernel(inp):
    rows, cols = inp.shape
    features, freqs, gains = _season_params(cols)
    featf = features.astype(jnp.float32).reshape(_N_FEATURES, 1)
    freqs = freqs.reshape(_N_FEATURES, 1)
    gains = gains.reshape(_N_FEATURES, 1)

    blk = 4096
    grid = rows // blk
    import functools

    return pl.pallas_call(
        functools.partial(_body, rows_total=rows),
        grid=(grid,),
        in_specs=[
            pl.BlockSpec((_N_FEATURES, 1), lambda i: (0, 0)),
            pl.BlockSpec((_N_FEATURES, 1), lambda i: (0, 0)),
            pl.BlockSpec((_N_FEATURES, 1), lambda i: (0, 0)),
            pl.BlockSpec((blk, cols), lambda i: (i, 0)),
        ],
        out_specs=pl.BlockSpec((blk, cols), lambda i: (i, 0)),
        out_shape=jax.ShapeDtypeStruct((rows, cols), jnp.float32),
    )(featf, freqs, gains, inp)


# speedup vs baseline: 7.9123x; 1.0006x over previous
import functools

import jax
import jax.numpy as jnp
from jax import lax
from jax.experimental import pallas as pl
from jax.experimental.pallas import tpu as pltpu

_N_FEATURES = 8
_FREQUENCY = 0.01
_GAIN = 1.0

_CHUNK = 1024
_NBUF = 8


def _season_params(num_cols):
    key = jax.random.key(42)
    key, kf = jax.random.split(key)
    features = jax.random.randint(kf, (_N_FEATURES,), 0, num_cols)
    freqs, gains = [], []
    for _ in range(_N_FEATURES):
        key, k1, k2 = jax.random.split(key, 3)
        freqs.append(jax.random.uniform(k1, (), dtype=jnp.float32) * _FREQUENCY)
        gains.append(jax.random.uniform(k2, (), dtype=jnp.float32) * _GAIN)
    return features, jnp.stack(freqs), jnp.stack(gains)


def _body(feat_ref, freq_ref, gain_ref, in_hbm, out_hbm, inb, outb, insem,
          outsem, *, rows_total, cols):
    ch, nb = _CHUNK, _NBUF
    nch = rows_total // ch

    def in_copy(c, b):
        return pltpu.make_async_copy(
            in_hbm.at[pl.ds(c * ch, ch), :], inb.at[b], insem.at[b]
        )

    def out_copy(c, b):
        return pltpu.make_async_copy(
            outb.at[b], out_hbm.at[pl.ds(c * ch, ch), :], outsem.at[b]
        )

    for b in range(nb):
        in_copy(b, b).start()

    col = jax.lax.broadcasted_iota(
        jnp.int32, (_N_FEATURES, cols), 1
    ).astype(jnp.float32)
    onehot = (col == feat_ref[...]).astype(jnp.float32)

    def superstep(s, carry):
        for b in range(nb):
            c = s * nb + b
            in_copy(c, b).wait()
            row = (
                jax.lax.broadcasted_iota(jnp.int32, (_N_FEATURES, ch), 1)
                + c * ch
            ).astype(jnp.float32)
            t = row * (1.0 / rows_total)
            x = t * 2.0 * 3.14 * freq_ref[...]
            season = gain_ref[...] * jnp.sin(x)

            @pl.when(c >= nb)
            def _():
                out_copy(c - nb, b).wait()

            outb[b] = inb[b] + jax.lax.dot_general(
                season,
                onehot,
                (((0,), (0,)), ((), ())),
                preferred_element_type=jnp.float32,
            )
            out_copy(c, b).start()

            @pl.when(c + nb < nch)
            def _():
                in_copy(c + nb, b).start()
        return carry

    lax.fori_loop(0, nch // nb, superstep, 0)
    for b in range(nb):
        out_copy(0, b).wait()


def kernel(inp):
    rows, cols = inp.shape
    features, freqs, gains = _season_params(cols)
    featf = features.astype(jnp.float32).reshape(_N_FEATURES, 1)
    freqs = freqs.reshape(_N_FEATURES, 1)
    gains = gains.reshape(_N_FEATURES, 1)

    return pl.pallas_call(
        functools.partial(_body, rows_total=rows, cols=cols),
        in_specs=[
            pl.BlockSpec((_N_FEATURES, 1), lambda: (0, 0)),
            pl.BlockSpec((_N_FEATURES, 1), lambda: (0, 0)),
            pl.BlockSpec((_N_FEATURES, 1), lambda: (0, 0)),
            pl.BlockSpec(memory_space=pl.ANY),
        ],
        out_specs=pl.BlockSpec(memory_space=pl.ANY),
        out_shape=jax.ShapeDtypeStruct((rows, cols), jnp.float32),
        scratch_shapes=[
            pltpu.VMEM((_NBUF, _CHUNK, cols), jnp.float32),
            pltpu.VMEM((_NBUF, _CHUNK, cols), jnp.float32),
            pltpu.SemaphoreType.DMA((_NBUF,)),
            pltpu.SemaphoreType.DMA((_NBUF,)),
        ],
    )(featf, freqs, gains, inp)
